# baseline (speedup 1.0000x reference)
import functools

import jax
import jax.numpy as jnp
from jax import lax
from jax.experimental import pallas as pl
from jax.experimental.pallas import tpu as pltpu
from jax.experimental.pallas import tpu_sc as plsc

_K = 1


def kernel(x, table):
    batch, seq_len = x.shape
    embed_dim = table.shape[1]
    info = plsc.get_sparse_core_info()
    nw = info.num_cores * info.num_subcores
    b_per_w = batch // nw
    n_dma = b_per_w // _K
    mesh = plsc.VectorSubcoreMesh(core_axis_name="c", subcore_axis_name="s")

    @functools.partial(
        pl.kernel,
        mesh=mesh,
        out_type=jax.ShapeDtypeStruct((batch, seq_len, embed_dim), table.dtype),
        scratch_types=[
            pltpu.VMEM((_K, seq_len, embed_dim), table.dtype),
            pltpu.SemaphoreType.DMA,
        ],
    )
    def _sc_bcast(table_hbm, out_hbm, buf, sem):
        wid = lax.axis_index("s") * info.num_cores + lax.axis_index("c")
        for k in range(_K):
            pltpu.sync_copy(table_hbm.at[pl.ds(0, seq_len)], buf.at[k])
        splits = [(0, 48), (48, 48), (96, 48), (144, seq_len - 144)]
        copies = [
            pltpu.async_copy(
                buf.at[:, pl.ds(off, sz)],
                out_hbm.at[pl.ds(wid * _K + j * _K * nw, _K), pl.ds(off, sz)],
                sem,
            )
            for j in range(n_dma)
            for (off, sz) in splits
        ]
        for c in copies:
            c.wait()

    return _sc_bcast(table)


# speedup vs baseline: 1.0175x; 1.0175x over previous
import functools

import jax
import jax.numpy as jnp
from jax import lax
from jax.experimental import pallas as pl
from jax.experimental.pallas import tpu as pltpu
from jax.experimental.pallas import tpu_sc as plsc

_K = 1


def kernel(x, table):
    batch, seq_len = x.shape
    embed_dim = table.shape[1]
    info = plsc.get_sparse_core_info()
    nw = info.num_cores * info.num_subcores
    b_per_w = batch // nw
    n_dma = b_per_w // _K
    mesh = plsc.VectorSubcoreMesh(core_axis_name="c", subcore_axis_name="s")

    @functools.partial(
        pl.kernel,
        mesh=mesh,
        out_type=jax.ShapeDtypeStruct((batch, seq_len, embed_dim), table.dtype),
        scratch_types=[
            pltpu.VMEM((_K, seq_len, embed_dim), table.dtype),
            pltpu.SemaphoreType.DMA,
        ],
    )
    def _sc_bcast(table_hbm, out_hbm, buf, sem):
        wid = lax.axis_index("s") * info.num_cores + lax.axis_index("c")
        for k in range(_K):
            pltpu.sync_copy(table_hbm.at[pl.ds(0, seq_len)], buf.at[k])
        splits = [(0, 96), (96, seq_len - 96)]
        copies = [
            pltpu.async_copy(
                buf.at[:, pl.ds(off, sz)],
                out_hbm.at[pl.ds(wid * _K + j * _K * nw, _K), pl.ds(off, sz)],
                sem,
            )
            for j in range(n_dma)
            for (off, sz) in splits
        ]
        for c in copies:
            c.wait()

    return _sc_bcast(table)


# speedup vs baseline: 1.0236x; 1.0060x over previous
import functools

import jax
import jax.numpy as jnp
from jax import lax
from jax.experimental import pallas as pl
from jax.experimental.pallas import tpu as pltpu
from jax.experimental.pallas import tpu_sc as plsc

_K = 1


def kernel(x, table):
    batch, seq_len = x.shape
    embed_dim = table.shape[1]
    info = plsc.get_sparse_core_info()
    nw = info.num_cores * info.num_subcores
    b_per_w = batch // nw
    n_dma = b_per_w // _K
    mesh = plsc.VectorSubcoreMesh(core_axis_name="c", subcore_axis_name="s")

    @functools.partial(
        pl.kernel,
        mesh=mesh,
        out_type=jax.ShapeDtypeStruct((batch, seq_len, embed_dim), table.dtype),
        scratch_types=[
            pltpu.VMEM((_K, seq_len, embed_dim), table.dtype),
            pltpu.SemaphoreType.DMA,
        ],
    )
    def _sc_bcast(table_hbm, out_hbm, buf, sem):
        wid = lax.axis_index("s") * info.num_cores + lax.axis_index("c")
        for k in range(_K):
            pltpu.sync_copy(table_hbm.at[pl.ds(0, seq_len)], buf.at[k])
        splits = [(0, 112), (112, seq_len - 112)]
        copies = [
            pltpu.async_copy(
                buf.at[:, pl.ds(off, sz)],
                out_hbm.at[pl.ds(wid * _K + j * _K * nw, _K), pl.ds(off, sz)],
                sem,
            )
            for j in range(n_dma)
            for (off, sz) in splits
        ]
        for c in copies:
            c.wait()

    return _sc_bcast(table)
